# baseline (speedup 1.0000x reference)
import functools

import jax
import jax.numpy as jnp
from jax.experimental import pallas as pl
from jax.experimental.pallas import tpu as pltpu

_N = 4096
_TM = 256
_GS = _N // _TM
_BETA = 0.5
_ALPHA = 1.0
_C = 0.72134752044448170367996234050095


def _unnorm_attn(ab_t, ah_col, bh_row):
    half = ab_t * (ah_col + bh_row)
    cb = jnp.bfloat16(_C)
    s = jnp.exp2(cb * jnp.tanh(half) + cb)
    return jnp.where(half != jnp.bfloat16(0.0), s, jnp.bfloat16(0.0))


def _ones_aug(h, tm):
    return jnp.concatenate([h, jnp.ones((tm, 1), jnp.float32)],
                           axis=1).astype(jnp.bfloat16)


def _recip_r(oa, k):
    r = oa[:, k:k + 1]
    return 1.0 / jnp.where(r == 0.0, 1.0, r)


def _b_row(v_ref, haug, k):
    b = jax.lax.dot_general(v_ref[...].astype(jnp.bfloat16), haug[:, :k],
                            (((0,), (1,)), ((), ())),
                            preferred_element_type=jnp.float32)
    return (0.5 * b).astype(jnp.bfloat16)


def _branch_kernel(with_head, x_ref, w1_ref, v11_ref, v12_ref, a_ref,
                   w2_ref, v21_ref, v22_ref, *rest):
    if with_head:
        (h1_ref, mu_ref, emb_ref, x_out_ref, hf_ref, q_ref,
         abf_scr, haug0_scr, haug1_scr, av_scr, d1_scr,
         t_scr, b0r_scr, b1r_scr) = rest
    else:
        (emb_ref, x_out_ref,
         abf_scr, haug0_scr, haug1_scr, av_scr, d1_scr,
         t_scr, b0r_scr, b1r_scr) = rest
    i = pl.program_id(1)
    k0 = haug0_scr.shape[1] - 1
    k1 = haug1_scr.shape[1] - 1
    rows = pl.ds(i * _TM, _TM)

    @pl.when(pl.program_id(0) == 0)
    def _proj():
        h = jnp.dot(x_ref[...], w1_ref[...],
                    preferred_element_type=jnp.float32)
        haug0_scr[rows, :] = _ones_aug(h, _TM)
        av_scr[rows, 0:1] = (0.5 * jnp.dot(h, v11_ref[...],
                                           preferred_element_type=jnp.float32)
                             ).astype(jnp.bfloat16)

    @pl.when((pl.program_id(0) == 1) & (i == 0))
    def _b0():
        b0r_scr[...] = _b_row(v12_ref, haug0_scr[...], k0)

    @pl.when((pl.program_id(0) == 2) & (i == 0))
    def _b1():
        b1r_scr[...] = _b_row(v22_ref, haug1_scr[...], k1)

    @pl.when(pl.program_id(0) == 1)
    def _enc1():
        ab_t = a_ref[...].astype(jnp.bfloat16)
        abf_scr[rows, :] = ab_t
        haug0 = haug0_scr[...]
        sb = _unnorm_attn(ab_t, av_scr[rows, 0:1], b0r_scr[...])
        oa = jnp.dot(sb, haug0, preferred_element_type=jnp.float32)
        ho = jnp.dot(oa[:, :k0] * _recip_r(oa, k0), w2_ref[...],
                     preferred_element_type=jnp.float32)
        haug1_scr[rows, :] = _ones_aug(ho, _TM)
        av_scr[rows, 1:2] = (0.5 * jnp.dot(ho, v21_ref[...],
                                           preferred_element_type=jnp.float32)
                             ).astype(jnp.bfloat16)

    @pl.when(pl.program_id(0) == 2)
    def _enc2():
        haug1 = haug1_scr[...]
        sb = _unnorm_attn(abf_scr[rows, :], av_scr[rows, 1:2], b1r_scr[...])
        oa = jnp.dot(sb, haug1, preferred_element_type=jnp.float32)
        o = oa[:, :k1] * _recip_r(oa, k1)
        emb_ref[...] = o
        d = jax.lax.dot_general(o, w2_ref[...], (((1,), (1,)), ((), ())),
                                preferred_element_type=jnp.float32)
        d1_scr[rows, :k0] = d.astype(jnp.bfloat16)
        d1_scr[rows, k0:k0 + 1] = jnp.ones((_TM, 1), jnp.bfloat16)
        d1_scr[rows, k0 + 1:k0 + 1 + k1] = o.astype(jnp.bfloat16)

    @pl.when(pl.program_id(0) == 3)
    def _dec1():
        sb = _unnorm_attn(abf_scr[rows, :], av_scr[rows, 1:2], b1r_scr[...])
        oa = jnp.dot(sb, d1_scr[:, :k0 + 1],
                     preferred_element_type=jnp.float32)
        t = oa[:, :k0] * _recip_r(oa, k0)
        t_scr[rows, :k0] = t.astype(jnp.bfloat16)
        t_scr[rows, k0:] = jnp.ones((_TM, 1), jnp.bfloat16)

    @pl.when(pl.program_id(0) == 4)
    def _dec2():
        sb = _unnorm_attn(abf_scr[rows, :], av_scr[rows, 0:1], b0r_scr[...])
        oa = jnp.dot(sb, t_scr[...], preferred_element_type=jnp.float32)
        o = oa[:, :k0] * _recip_r(oa, k0)
        x_out_ref[...] = jax.lax.dot_general(
            o, w1_ref[...], (((1,), (1,)), ((), ())),
            preferred_element_type=jnp.float32)
        if with_head:
            hf = (h1_ref[...]
                  + _BETA * d1_scr[rows, k0 + 1:k0 + 1 + k1].astype(
                      jnp.float32))
            hf_ref[...] = hf
            mu = mu_ref[...]
            hn = jnp.sum(hf * hf, axis=1, keepdims=True)
            mn = jnp.sum(mu * mu, axis=1)[None, :]
            cross = jax.lax.dot_general(hf, mu, (((1,), (1,)), ((), ())),
                                        preferred_element_type=jnp.float32)
            d2 = hn + mn - 2.0 * cross
            qun = (1.0 + d2 / _ALPHA) ** (-(_ALPHA + 1.0) / 2.0)
            q_ref[...] = qun / jnp.sum(qun, axis=1, keepdims=True)


def _branch(A, X, W1, v11, v12, W2, v21, v22, head=None):
    k_in, k0 = W1.shape
    k1 = W2.shape[1]
    in_specs = [
        pl.BlockSpec((_TM, k_in),
                     lambda p, i: (jnp.where(p == 0, i, _GS - 1), 0)),
        pl.BlockSpec((k_in, k0), lambda p, i: (0, 0)),
        pl.BlockSpec((k0, 1), lambda p, i: (0, 0)),
        pl.BlockSpec((k0, 1), lambda p, i: (0, 0)),
        pl.BlockSpec((_TM, _N), lambda p, i: (
            jnp.where(p == 1, i, jnp.where(p == 0, 0, _GS - 1)), 0)),
        pl.BlockSpec((k0, k1), lambda p, i: (0, 0)),
        pl.BlockSpec((k1, 1), lambda p, i: (0, 0)),
        pl.BlockSpec((k1, 1), lambda p, i: (0, 0)),
    ]
    args = [X, W1, v11, v12, A, W2, v21, v22]
    out_specs = [
        pl.BlockSpec((_TM, k1), lambda p, i: (jnp.where(p == 2, i, 0), 0)),
        pl.BlockSpec((_TM, k_in),
                     lambda p, i: (jnp.where(p == 4, i, 0), 0)),
    ]
    out_shape = [
        jax.ShapeDtypeStruct((_N, k1), jnp.float32),
        jax.ShapeDtypeStruct((_N, k_in), jnp.float32),
    ]
    if head is not None:
        H1, mu = head
        nc = mu.shape[0]
        in_specs += [
            pl.BlockSpec((_TM, k1),
                         lambda p, i: (jnp.where(p == 4, i, 0), 0)),
            pl.BlockSpec((nc, k1), lambda p, i: (0, 0)),
        ]
        args += [H1, mu]
        out_specs += [
            pl.BlockSpec((_TM, k1),
                         lambda p, i: (jnp.where(p == 4, i, 0), 0)),
            pl.BlockSpec((_TM, nc),
                         lambda p, i: (jnp.where(p == 4, i, 0), 0)),
        ]
        out_shape += [
            jax.ShapeDtypeStruct((_N, k1), jnp.float32),
            jax.ShapeDtypeStruct((_N, nc), jnp.float32),
        ]
    body = functools.partial(_branch_kernel, head is not None)
    return pl.pallas_call(
        body,
        grid=(5, _GS),
        in_specs=in_specs,
        out_specs=out_specs,
        out_shape=out_shape,
        scratch_shapes=[
            pltpu.VMEM((_N, _N), jnp.bfloat16),
            pltpu.VMEM((_N, k0 + 1), jnp.bfloat16),
            pltpu.VMEM((_N, k1 + 1), jnp.bfloat16),
            pltpu.VMEM((_N, 2), jnp.bfloat16),
            pltpu.VMEM((_N, k0 + 1 + k1), jnp.bfloat16),
            pltpu.VMEM((_N, k0 + 1), jnp.bfloat16),
            pltpu.VMEM((1, _N), jnp.bfloat16),
            pltpu.VMEM((1, _N), jnp.bfloat16),
        ],
        compiler_params=pltpu.CompilerParams(
            dimension_semantics=("arbitrary", "arbitrary")),
    )(*args)


def kernel(A, X, A2, X2, W11, v111, v112, W12, v121, v122, W21, v211, v212,
           W22, v221, v222, mu):
    H1, X_ = _branch(A, X, W11, v111, v112, W12, v121, v122)
    H2, X_2, H_F, q = _branch(A2, X2, W21, v211, v212, W22, v221, v222,
                              head=(H1, mu))
    return (H_F, q, H1, H2, X_, X_2)


# speedup vs baseline: 1.2179x; 1.2179x over previous
import functools

import jax
import jax.numpy as jnp
from jax.experimental import pallas as pl
from jax.experimental.pallas import tpu as pltpu

_N = 4096
_TM = 512
_GS = _N // _TM
_BETA = 0.5
_ALPHA = 1.0
_C = 0.72134752044448170367996234050095


def _unnorm_attn(ab_t, ah_col, bh_row):
    half = ab_t * (ah_col + bh_row)
    cb = jnp.bfloat16(_C)
    s = jnp.exp2(cb * jnp.tanh(half) + cb)
    return jnp.where(half != jnp.bfloat16(0.0), s, jnp.bfloat16(0.0))


def _ones_aug(h, tm):
    return jnp.concatenate([h, jnp.ones((tm, 1), jnp.float32)],
                           axis=1).astype(jnp.bfloat16)


def _recip_r(oa, k):
    r = oa[:, k:k + 1]
    return 1.0 / jnp.where(r == 0.0, 1.0, r)


def _b_row(v_ref, haug, k):
    b = jax.lax.dot_general(v_ref[...].astype(jnp.bfloat16), haug[:, :k],
                            (((0,), (1,)), ((), ())),
                            preferred_element_type=jnp.float32)
    return (0.5 * b).astype(jnp.bfloat16)


def _branch_kernel(with_head, x_ref, w1_ref, v11_ref, v12_ref, a_ref,
                   w2_ref, v21_ref, v22_ref, *rest):
    if with_head:
        (h1_ref, mu_ref, emb_ref, x_out_ref, hf_ref, q_ref,
         abf_scr, haug0_scr, haug1_scr, av_scr, d1_scr,
         t_scr, b0r_scr, b1r_scr) = rest
    else:
        (emb_ref, x_out_ref,
         abf_scr, haug0_scr, haug1_scr, av_scr, d1_scr,
         t_scr, b0r_scr, b1r_scr) = rest
    i = pl.program_id(1)
    k0 = haug0_scr.shape[1] - 1
    k1 = haug1_scr.shape[1] - 1
    rows = pl.ds(i * _TM, _TM)

    @pl.when((pl.program_id(0) == 0) & (i == 0))
    def _proj():
        h = jnp.dot(x_ref[...], w1_ref[...],
                    preferred_element_type=jnp.float32)
        haug0_scr[...] = _ones_aug(h, _N)
        av_scr[:, 0:1] = (0.5 * jnp.dot(h, v11_ref[...],
                                        preferred_element_type=jnp.float32)
                          ).astype(jnp.bfloat16)

    @pl.when((pl.program_id(0) == 1) & (i == 0))
    def _b0():
        b0r_scr[...] = _b_row(v12_ref, haug0_scr[...], k0)

    @pl.when((pl.program_id(0) == 2) & (i == 0))
    def _b1():
        b1r_scr[...] = _b_row(v22_ref, haug1_scr[...], k1)

    @pl.when(pl.program_id(0) == 1)
    def _enc1():
        ab_t = a_ref[...].astype(jnp.bfloat16)
        abf_scr[rows, :] = ab_t
        haug0 = haug0_scr[...]
        sb = _unnorm_attn(ab_t, av_scr[rows, 0:1], b0r_scr[...])
        oa = jnp.dot(sb, haug0, preferred_element_type=jnp.float32)
        ho = jnp.dot(oa[:, :k0] * _recip_r(oa, k0), w2_ref[...],
                     preferred_element_type=jnp.float32)
        haug1_scr[rows, :] = _ones_aug(ho, _TM)
        av_scr[rows, 1:2] = (0.5 * jnp.dot(ho, v21_ref[...],
                                           preferred_element_type=jnp.float32)
                             ).astype(jnp.bfloat16)

    @pl.when(pl.program_id(0) == 2)
    def _enc2():
        haug1 = haug1_scr[...]
        sb = _unnorm_attn(abf_scr[rows, :], av_scr[rows, 1:2], b1r_scr[...])
        oa = jnp.dot(sb, haug1, preferred_element_type=jnp.float32)
        o = oa[:, :k1] * _recip_r(oa, k1)
        emb_ref[...] = o
        d = jax.lax.dot_general(o, w2_ref[...], (((1,), (1,)), ((), ())),
                                preferred_element_type=jnp.float32)
        d1_scr[rows, :k0] = d.astype(jnp.bfloat16)
        d1_scr[rows, k0:k0 + 1] = jnp.ones((_TM, 1), jnp.bfloat16)
        d1_scr[rows, k0 + 1:k0 + 1 + k1] = o.astype(jnp.bfloat16)

    @pl.when(pl.program_id(0) == 3)
    def _dec1():
        sb = _unnorm_attn(abf_scr[rows, :], av_scr[rows, 1:2], b1r_scr[...])
        oa = jnp.dot(sb, d1_scr[:, :k0 + 1],
                     preferred_element_type=jnp.float32)
        t = oa[:, :k0] * _recip_r(oa, k0)
        t_scr[rows, :k0] = t.astype(jnp.bfloat16)
        t_scr[rows, k0:] = jnp.ones((_TM, 1), jnp.bfloat16)

    @pl.when(pl.program_id(0) == 4)
    def _dec2():
        sb = _unnorm_attn(abf_scr[rows, :], av_scr[rows, 0:1], b0r_scr[...])
        oa = jnp.dot(sb, t_scr[...], preferred_element_type=jnp.float32)
        o = oa[:, :k0] * _recip_r(oa, k0)
        x_out_ref[...] = jax.lax.dot_general(
            o, w1_ref[...], (((1,), (1,)), ((), ())),
            preferred_element_type=jnp.float32)
        if with_head:
            hf = (h1_ref[...]
                  + _BETA * d1_scr[rows, k0 + 1:k0 + 1 + k1].astype(
                      jnp.float32))
            hf_ref[...] = hf
            mu = mu_ref[...]
            hn = jnp.sum(hf * hf, axis=1, keepdims=True)
            mn = jnp.sum(mu * mu, axis=1)[None, :]
            cross = jax.lax.dot_general(hf, mu, (((1,), (1,)), ((), ())),
                                        preferred_element_type=jnp.float32)
            d2 = hn + mn - 2.0 * cross
            qun = (1.0 + d2 / _ALPHA) ** (-(_ALPHA + 1.0) / 2.0)
            q_ref[...] = qun / jnp.sum(qun, axis=1, keepdims=True)


def _branch(A, X, W1, v11, v12, W2, v21, v22, head=None):
    k_in, k0 = W1.shape
    k1 = W2.shape[1]
    in_specs = [
        pl.BlockSpec((_N, k_in), lambda p, i: (0, 0)),
        pl.BlockSpec((k_in, k0), lambda p, i: (0, 0)),
        pl.BlockSpec((k0, 1), lambda p, i: (0, 0)),
        pl.BlockSpec((k0, 1), lambda p, i: (0, 0)),
        pl.BlockSpec((_TM, _N), lambda p, i: (
            jnp.where(p == 1, i, jnp.where(p == 0, 0, _GS - 1)), 0)),
        pl.BlockSpec((k0, k1), lambda p, i: (0, 0)),
        pl.BlockSpec((k1, 1), lambda p, i: (0, 0)),
        pl.BlockSpec((k1, 1), lambda p, i: (0, 0)),
    ]
    args = [X, W1, v11, v12, A, W2, v21, v22]
    out_specs = [
        pl.BlockSpec((_TM, k1), lambda p, i: (jnp.where(p == 2, i, 0), 0)),
        pl.BlockSpec((_TM, k_in),
                     lambda p, i: (jnp.where(p == 4, i, 0), 0)),
    ]
    out_shape = [
        jax.ShapeDtypeStruct((_N, k1), jnp.float32),
        jax.ShapeDtypeStruct((_N, k_in), jnp.float32),
    ]
    if head is not None:
        H1, mu = head
        nc = mu.shape[0]
        in_specs += [
            pl.BlockSpec((_TM, k1),
                         lambda p, i: (jnp.where(p == 4, i, 0), 0)),
            pl.BlockSpec((nc, k1), lambda p, i: (0, 0)),
        ]
        args += [H1, mu]
        out_specs += [
            pl.BlockSpec((_TM, k1),
                         lambda p, i: (jnp.where(p == 4, i, 0), 0)),
            pl.BlockSpec((_TM, nc),
                         lambda p, i: (jnp.where(p == 4, i, 0), 0)),
        ]
        out_shape += [
            jax.ShapeDtypeStruct((_N, k1), jnp.float32),
            jax.ShapeDtypeStruct((_N, nc), jnp.float32),
        ]
    body = functools.partial(_branch_kernel, head is not None)
    return pl.pallas_call(
        body,
        grid=(5, _GS),
        in_specs=in_specs,
        out_specs=out_specs,
        out_shape=out_shape,
        scratch_shapes=[
            pltpu.VMEM((_N, _N), jnp.bfloat16),
            pltpu.VMEM((_N, k0 + 1), jnp.bfloat16),
            pltpu.VMEM((_N, k1 + 1), jnp.bfloat16),
            pltpu.VMEM((_N, 2), jnp.bfloat16),
            pltpu.VMEM((_N, k0 + 1 + k1), jnp.bfloat16),
            pltpu.VMEM((_N, k0 + 1), jnp.bfloat16),
            pltpu.VMEM((1, _N), jnp.bfloat16),
            pltpu.VMEM((1, _N), jnp.bfloat16),
        ],
        compiler_params=pltpu.CompilerParams(
            dimension_semantics=("arbitrary", "arbitrary")),
    )(*args)


def kernel(A, X, A2, X2, W11, v111, v112, W12, v121, v122, W21, v211, v212,
           W22, v221, v222, mu):
    H1, X_ = _branch(A, X, W11, v111, v112, W12, v121, v122)
    H2, X_2, H_F, q = _branch(A2, X2, W21, v211, v212, W22, v221, v222,
                              head=(H1, mu))
    return (H_F, q, H1, H2, X_, X_2)
